# Initial kernel scaffold; baseline (speedup 1.0000x reference)
import jax
import jax.numpy as jnp
from jax.experimental import pallas as pl


def kernel(x, edge_index, edge_attr, batch, W, b):
    raise NotImplementedError("write your pallas kernel here")


# speedup vs baseline: 77.7308x; 77.7308x over previous
import functools

import jax
import jax.numpy as jnp
from jax import lax
from jax.experimental import pallas as pl
from jax.experimental.pallas import tpu as pltpu, tpu_sc as plsc

N = 10000
E = 320000
D = 128
G = 64
RATIO = 0.8

NS = 16
NP = 10240
NPT = NP // NS
ECH = 128
ECHUNKS = 157
EPT = ECH * ECHUNKS
EP = EPT * NS
GPT = G // NS
NEG_INF = -3.0e38


def _matvec_body(x_ref, w_ref, o_ref):
    o_ref[...] = jnp.dot(x_ref[...], w_ref[...],
                         preferred_element_type=jnp.float32)


def _matvec(x, W):
    return pl.pallas_call(
        _matvec_body,
        out_shape=jax.ShapeDtypeStruct((N, 1), jnp.float32),
    )(x, W)


def _lane():
    return lax.iota(jnp.int32, 16)


def _extract_f32(vec, j):
    return jnp.sum(jnp.where(_lane() == j, vec, jnp.float32(0.0)))


def _extract_i32(vec, j):
    return jnp.sum(jnp.where(_lane() == j, vec, jnp.int32(0)))


def _sortable_u32(s16):
    b = lax.bitcast_convert_type(s16, jnp.int32)
    m = (b >> 31) | jnp.int32(-2147483648)
    return lax.bitcast_convert_type(b ^ m, jnp.uint32)


def _newton_rsqrt(xf):
    i = lax.bitcast_convert_type(xf, jnp.int32)
    y = lax.bitcast_convert_type(jnp.int32(0x5F3759DF) - (i >> 1), jnp.float32)
    for _ in range(4):
        y = y * (jnp.float32(1.5) - jnp.float32(0.5) * xf * y * y)
    return y


def _tanh16(s16):
    e = jnp.exp(jnp.float32(2.0) * s16)
    return jnp.float32(1.0) - jnp.float32(2.0) / (e + jnp.float32(1.0))


def _sc_body(hp, src3, dst3, w3, batch3, b16, x_hbm, out_hbm,
             deg_sh, dinv_sh, score_sh, cnt_sh,
             srcb, dstb, wb, batchb, nodeb, hb, db, scv,
             cntb, offb, onesb, rows, outrow, bb):
    wid = lax.axis_index("s")
    nbase = wid * NPT

    zero16 = jnp.zeros((16,), jnp.float32)
    one16 = jnp.ones((16,), jnp.float32)
    for c in range(NPT // 16):
        nodeb[pl.ds(c * 16, 16)] = zero16
    pltpu.sync_copy(nodeb, deg_sh.at[pl.ds(nbase, NPT)])

    @pl.when(wid == 0)
    def _():
        for c in range(5):
            cntb[pl.ds(c * 16, 16)] = zero16
        pltpu.sync_copy(cntb, cnt_sh)

    for c in range(8):
        onesb[pl.ds(c * 16, 16)] = one16

    pltpu.sync_copy(src3.at[wid], srcb)
    pltpu.sync_copy(dst3.at[wid], dstb)
    pltpu.sync_copy(w3.at[wid], wb)
    pltpu.sync_copy(batch3.at[wid], batchb)
    pltpu.sync_copy(b16, bb)
    plsc.subcore_barrier()

    def _deg_chunk(c, carry):
        pltpu.sync_copy(wb.at[c], deg_sh.at[dstb.at[c]], add=True)
        return carry
    lax.fori_loop(0, ECHUNKS, _deg_chunk, 0)
    for j in range(5):
        pltpu.sync_copy(onesb.at[pl.ds(0, 128)],
                        cnt_sh.at[batchb.at[j]], add=True)
    plsc.subcore_barrier()

    pltpu.sync_copy(deg_sh.at[pl.ds(nbase, NPT)], nodeb)
    for c in range(NPT // 16):
        dg = nodeb[pl.ds(c * 16, 16)] + jnp.float32(1.0)
        nodeb[pl.ds(c * 16, 16)] = _newton_rsqrt(dg)
    pltpu.sync_copy(nodeb, dinv_sh.at[pl.ds(nbase, NPT)])
    plsc.subcore_barrier()

    pltpu.sync_copy(hp, hb)
    pltpu.sync_copy(dinv_sh, db)
    bvec = bb[...]
    for c in range(NPT // 16):
        hv = hb[pl.ds(nbase + c * 16, 16)]
        dv = db[pl.ds(nbase + c * 16, 16)]
        nodeb[pl.ds(c * 16, 16)] = hv * dv * dv + bvec
    pltpu.sync_copy(nodeb, score_sh.at[pl.ds(nbase, NPT)])
    plsc.subcore_barrier()

    def _val_chunk(c, carry):
        for l in range(8):
            sl = pl.ds(l * 16, 16)
            s16 = srcb[c, sl]
            d16 = dstb[c, sl]
            w16 = wb[c, sl]
            hs = plsc.load_gather(hb, [s16])
            dvs = plsc.load_gather(db, [s16])
            dvd = plsc.load_gather(db, [d16])
            wb[c, sl] = w16 * hs * dvs * dvd
        return carry
    lax.fori_loop(0, ECHUNKS, _val_chunk, 0)

    def _sc_chunk(c, carry):
        pltpu.sync_copy(wb.at[c], score_sh.at[dstb.at[c]], add=True)
        return carry
    lax.fori_loop(0, ECHUNKS, _sc_chunk, 0)
    plsc.subcore_barrier()

    pltpu.sync_copy(score_sh, scv)
    pltpu.sync_copy(cnt_sh, cntb)
    carry0 = jnp.zeros((16,), jnp.int32)
    for c in range(4):
        cf = cntb[pl.ds(c * 16, 16)]
        ci = cf.astype(jnp.int32)
        incl = plsc.cumsum(ci)
        offb[pl.ds(c * 16, 16)] = incl - ci + carry0
        carry0 = carry0 + jnp.broadcast_to(_extract_i32(incl, 15), (16,))

    lane = _lane()

    for j in range(GPT):
        g = wid * GPT + j
        gchunk = (g >> 4) * 16
        glane = g & 15
        off16 = offb[pl.ds(gchunk, 16)]
        cnt16 = cntb[pl.ds(gchunk, 16)]
        start = _extract_i32(off16, glane)
        cf = _extract_f32(cnt16, glane)
        cnt = cf.astype(jnp.int32)
        end = start + cnt
        prod = jnp.float32(RATIO) * cf
        tk = prod.astype(jnp.int32)
        kk = tk + jnp.where(tk.astype(jnp.float32) < prod,
                            jnp.int32(1), jnp.int32(0))
        k16 = jnp.broadcast_to(kk, (16,))

        b0 = (start >> 4) << 4
        nchunks = (end - b0 + 15) >> 4

        def _count_gt(t16):
            def _cc(i, acc):
                nb = b0 + i * 16
                s16 = scv[pl.ds(nb, 16)]
                u = _sortable_u32(s16)
                nidx = lane + nb
                m = (u > t16) & (nidx >= start) & (nidx < end)
                return acc + plsc.all_reduce_population_count(m)
            return lax.fori_loop(0, nchunks, _cc, jnp.zeros((16,), jnp.int32))

        def _bs(i, lh):
            lo, hi = lh
            mid = lo + ((hi - lo) >> jnp.uint32(1))
            cgt = _count_gt(mid)
            pred = cgt >= k16
            lo2 = jnp.where(pred, mid + jnp.uint32(1), lo)
            hi2 = jnp.where(pred, hi, mid)
            return (lo2, hi2)
        v16, _ = lax.fori_loop(
            0, 32, _bs,
            (jnp.zeros((16,), jnp.uint32),
             jnp.full((16,), jnp.uint32(0xFFFFFFFF))))
        ngt = _count_gt(v16)
        slots16 = k16 - ngt

        acc0 = tuple(jnp.zeros((16,), jnp.float32) for _ in range(8)) + \
               tuple(jnp.full((16,), NEG_INF, jnp.float32) for _ in range(8)) + \
               (jnp.zeros((16,), jnp.int32),)

        def _pool(i, acc):
            tie = acc[16]
            nb = b0 + i * 16
            s16 = scv[pl.ds(nb, 16)]
            u = _sortable_u32(s16)
            nidx = lane + nb
            inr = (nidx >= start) & (nidx < end)
            eq = ((u == v16) & inr).astype(jnp.int32)
            texc = plsc.cumsum(eq) - eq + tie
            keep = inr & ((u > v16) | ((u == v16) & (texc < slots16)))
            tie = tie + jnp.broadcast_to(jnp.sum(eq), (16,))
            th = _tanh16(jnp.where(inr, s16, jnp.float32(0.0)))
            wsum = jnp.where(keep, th, jnp.float32(0.0))
            pen = jnp.where(keep, jnp.float32(0.0), jnp.float32(NEG_INF))
            pltpu.sync_copy(x_hbm.at[pl.ds(pl.multiple_of(nb, 16), 16)], rows)
            news = list(acc[0:8])
            newm = list(acc[8:16])
            for node in range(16):
                tj = _extract_f32(th, node)
                wj = _extract_f32(wsum, node)
                pj = _extract_f32(pen, node)
                for q in range(8):
                    row = rows[node, pl.ds(q * 16, 16)]
                    news[q] = news[q] + row * wj
                    newm[q] = jnp.maximum(newm[q], row * tj + pj)
            return tuple(news) + tuple(newm) + (tie,)

        accf = lax.fori_loop(0, nchunks, _pool, acc0)
        denom = jnp.maximum(kk.astype(jnp.float32), jnp.float32(1.0))
        for q in range(8):
            outrow[pl.ds(q * 16, 16)] = accf[q] / denom
            mx = accf[8 + q]
            outrow[pl.ds(128 + q * 16, 16)] = jnp.where(
                mx > NEG_INF, mx, jnp.float32(0.0))
        pltpu.sync_copy(outrow, out_hbm.at[g])


@jax.jit
def kernel(x, edge_index, edge_attr, batch, W, b):
    h = _matvec(x, W)[:, 0]
    hp = jnp.pad(h, (0, NP - N))
    src = jnp.pad(edge_index[0], (0, EP - E)).reshape(NS, ECHUNKS, ECH)
    dst = jnp.pad(edge_index[1], (0, EP - E)).reshape(NS, ECHUNKS, ECH)
    w = jnp.pad(jnp.ones((E,), jnp.float32),
                (0, EP - E)).reshape(NS, ECHUNKS, ECH)
    bat = jnp.pad(batch, (0, NP - N), constant_values=G).reshape(NS, 5, 128)
    b16 = jnp.broadcast_to(b, (16,)).astype(jnp.float32)

    mesh = plsc.VectorSubcoreMesh(core_axis_name="c", subcore_axis_name="s",
                                  num_cores=1)
    out = pl.kernel(
        _sc_body,
        out_type=jax.ShapeDtypeStruct((G, 2 * D), jnp.float32),
        mesh=mesh,
        compiler_params=pltpu.CompilerParams(needs_layout_passes=False),
        scratch_types=[
            pltpu.VMEM_SHARED((NP,), jnp.float32),
            pltpu.VMEM_SHARED((NP,), jnp.float32),
            pltpu.VMEM_SHARED((NP,), jnp.float32),
            pltpu.VMEM_SHARED((80,), jnp.float32),
            pltpu.VMEM((ECHUNKS, ECH), jnp.int32),
            pltpu.VMEM((ECHUNKS, ECH), jnp.int32),
            pltpu.VMEM((ECHUNKS, ECH), jnp.float32),
            pltpu.VMEM((5, 128), jnp.int32),
            pltpu.VMEM((NPT,), jnp.float32),
            pltpu.VMEM((NP,), jnp.float32),
            pltpu.VMEM((NP,), jnp.float32),
            pltpu.VMEM((NP,), jnp.float32),
            pltpu.VMEM((80,), jnp.float32),
            pltpu.VMEM((80,), jnp.int32),
            pltpu.VMEM((128,), jnp.float32),
            pltpu.VMEM((16, D), jnp.float32),
            pltpu.VMEM((2 * D,), jnp.float32),
            pltpu.VMEM((16,), jnp.float32),
        ],
    )(hp, src, dst, w, bat, b16, x)
    return out


# speedup vs baseline: 85.2856x; 1.0972x over previous
import functools

import jax
import jax.numpy as jnp
from jax import lax
from jax.experimental import pallas as pl
from jax.experimental.pallas import tpu as pltpu, tpu_sc as plsc

N = 10000
E = 320000
D = 128
G = 64
RATIO = 0.8

NS = 16
NP = 10240
NPT = NP // NS
ECH = 128
ECHUNKS = 157
EPT = ECH * ECHUNKS
EP = EPT * NS
GPT = G // NS
NEG_INF = -3.0e38


def _matvec_body(x_ref, w_ref, o_ref):
    o_ref[...] = jnp.dot(x_ref[...], w_ref[...],
                         preferred_element_type=jnp.float32)


def _matvec(x, W):
    return pl.pallas_call(
        _matvec_body,
        out_shape=jax.ShapeDtypeStruct((N, 1), jnp.float32),
    )(x, W)


def _lane():
    return lax.iota(jnp.int32, 16)


def _extract_f32(vec, j):
    return jnp.sum(jnp.where(_lane() == j, vec, jnp.float32(0.0)))


def _extract_i32(vec, j):
    return jnp.sum(jnp.where(_lane() == j, vec, jnp.int32(0)))


def _sortable_u32(s16):
    b = lax.bitcast_convert_type(s16, jnp.int32)
    m = (b >> 31) | jnp.int32(-2147483648)
    return lax.bitcast_convert_type(b ^ m, jnp.uint32)


def _newton_rsqrt(xf):
    i = lax.bitcast_convert_type(xf, jnp.int32)
    y = lax.bitcast_convert_type(jnp.int32(0x5F3759DF) - (i >> 1), jnp.float32)
    for _ in range(4):
        y = y * (jnp.float32(1.5) - jnp.float32(0.5) * xf * y * y)
    return y


def _tanh16(s16):
    e = jnp.exp(jnp.float32(2.0) * s16)
    return jnp.float32(1.0) - jnp.float32(2.0) / (e + jnp.float32(1.0))


def _sc_body(hp, src3, dst3, w3, batch3, b16, x_hbm, out_hbm,
             deg_sh, dinv_sh, score_sh, cnt_sh,
             srcb, dstb, wb, batchb, nodeb, hb, db, scv,
             cntb, offb, onesb, rows, outrow, bb, sem):
    wid = lax.axis_index("s")
    nbase = wid * NPT

    zero16 = jnp.zeros((16,), jnp.float32)
    one16 = jnp.ones((16,), jnp.float32)
    for c in range(NPT // 16):
        nodeb[pl.ds(c * 16, 16)] = zero16
    pltpu.sync_copy(nodeb, deg_sh.at[pl.ds(nbase, NPT)])

    @pl.when(wid == 0)
    def _():
        for c in range(5):
            cntb[pl.ds(c * 16, 16)] = zero16
        pltpu.sync_copy(cntb, cnt_sh)

    for c in range(8):
        onesb[pl.ds(c * 16, 16)] = one16

    stage = [pltpu.async_copy(src3.at[wid], srcb, sem),
             pltpu.async_copy(dst3.at[wid], dstb, sem),
             pltpu.async_copy(w3.at[wid], wb, sem),
             pltpu.async_copy(batch3.at[wid], batchb, sem),
             pltpu.async_copy(b16, bb, sem)]
    for d in stage:
        d.wait()
    plsc.subcore_barrier()

    def _deg_chunk(c, carry):
        pltpu.sync_copy(wb.at[c], deg_sh.at[dstb.at[c]], add=True)
        return carry
    lax.fori_loop(0, ECHUNKS, _deg_chunk, 0)
    for j in range(5):
        pltpu.sync_copy(onesb.at[pl.ds(0, 128)],
                        cnt_sh.at[batchb.at[j]], add=True)
    plsc.subcore_barrier()

    pltpu.sync_copy(deg_sh.at[pl.ds(nbase, NPT)], nodeb)
    for c in range(NPT // 16):
        dg = nodeb[pl.ds(c * 16, 16)] + jnp.float32(1.0)
        nodeb[pl.ds(c * 16, 16)] = _newton_rsqrt(dg)
    pltpu.sync_copy(nodeb, dinv_sh.at[pl.ds(nbase, NPT)])
    plsc.subcore_barrier()

    pltpu.sync_copy(hp, hb)
    pltpu.sync_copy(dinv_sh, db)
    bvec = bb[...]
    for c in range(NPT // 16):
        hv = hb[pl.ds(nbase + c * 16, 16)]
        dv = db[pl.ds(nbase + c * 16, 16)]
        nodeb[pl.ds(c * 16, 16)] = hv * dv * dv + bvec
    pltpu.sync_copy(nodeb, score_sh.at[pl.ds(nbase, NPT)])
    plsc.subcore_barrier()

    def _val_chunk(c, carry):
        for l in range(8):
            sl = pl.ds(l * 16, 16)
            s16 = srcb[c, sl]
            d16 = dstb[c, sl]
            w16 = wb[c, sl]
            hs = plsc.load_gather(hb, [s16])
            dvs = plsc.load_gather(db, [s16])
            dvd = plsc.load_gather(db, [d16])
            wb[c, sl] = w16 * hs * dvs * dvd
        return carry
    lax.fori_loop(0, ECHUNKS, _val_chunk, 0)

    def _sc_chunk(c, carry):
        pltpu.sync_copy(wb.at[c], score_sh.at[dstb.at[c]], add=True)
        return carry
    lax.fori_loop(0, ECHUNKS, _sc_chunk, 0)
    plsc.subcore_barrier()

    pltpu.sync_copy(score_sh, scv)
    pltpu.sync_copy(cnt_sh, cntb)
    carry0 = jnp.zeros((16,), jnp.int32)
    for c in range(4):
        cf = cntb[pl.ds(c * 16, 16)]
        ci = cf.astype(jnp.int32)
        incl = plsc.cumsum(ci)
        offb[pl.ds(c * 16, 16)] = incl - ci + carry0
        carry0 = carry0 + jnp.broadcast_to(_extract_i32(incl, 15), (16,))

    lane = _lane()

    for j in range(GPT):
        g = wid * GPT + j
        gchunk = (g >> 4) * 16
        glane = g & 15
        off16 = offb[pl.ds(gchunk, 16)]
        cnt16 = cntb[pl.ds(gchunk, 16)]
        start = _extract_i32(off16, glane)
        cf = _extract_f32(cnt16, glane)
        cnt = cf.astype(jnp.int32)
        end = start + cnt
        prod = jnp.float32(RATIO) * cf
        tk = prod.astype(jnp.int32)
        kk = tk + jnp.where(tk.astype(jnp.float32) < prod,
                            jnp.int32(1), jnp.int32(0))
        k16 = jnp.broadcast_to(kk, (16,))

        b0 = (start >> 4) << 4
        nchunks = (end - b0 + 15) >> 4

        def _count_gt(t16):
            def _cc(i, acc):
                nb = b0 + i * 16
                s16 = scv[pl.ds(nb, 16)]
                u = _sortable_u32(s16)
                nidx = lane + nb
                m = (u > t16) & (nidx >= start) & (nidx < end)
                return acc + plsc.all_reduce_population_count(m)
            return lax.fori_loop(0, nchunks, _cc, jnp.zeros((16,), jnp.int32))

        def _bs(i, lh):
            lo, hi = lh
            mid = lo + ((hi - lo) >> jnp.uint32(1))
            cgt = _count_gt(mid)
            pred = cgt >= k16
            lo2 = jnp.where(pred, mid + jnp.uint32(1), lo)
            hi2 = jnp.where(pred, hi, mid)
            return (lo2, hi2)
        v16, _ = lax.fori_loop(
            0, 32, _bs,
            (jnp.zeros((16,), jnp.uint32),
             jnp.full((16,), jnp.uint32(0xFFFFFFFF))))
        ngt = _count_gt(v16)
        slots16 = k16 - ngt

        acc0 = tuple(jnp.zeros((16,), jnp.float32) for _ in range(8)) + \
               tuple(jnp.full((16,), NEG_INF, jnp.float32) for _ in range(8)) + \
               (jnp.zeros((16,), jnp.int32),)

        mcount = (nchunks + 3) >> 2

        def _pool(m, acc):
            descs = []
            for j in range(4):
                nb = b0 + (m * 4 + j) * 16
                nbc = jnp.minimum(nb, jnp.int32(N - 16))
                descs.append(pltpu.async_copy(
                    x_hbm.at[pl.ds(pl.multiple_of(nbc, 16), 16)],
                    rows.at[pl.ds(j * 16, 16)], sem))
            for d in descs:
                d.wait()
            for j in range(4):
                tie = acc[16]
                nb = b0 + (m * 4 + j) * 16
                s16 = scv[pl.ds(nb, 16)]
                u = _sortable_u32(s16)
                nidx = lane + nb
                inr = (nidx >= start) & (nidx < end)
                eq = ((u == v16) & inr).astype(jnp.int32)
                texc = plsc.cumsum(eq) - eq + tie
                keep = inr & ((u > v16) | ((u == v16) & (texc < slots16)))
                tie = tie + jnp.broadcast_to(jnp.sum(eq), (16,))
                th = _tanh16(jnp.where(inr, s16, jnp.float32(0.0)))
                wsum = jnp.where(keep, th, jnp.float32(0.0))
                pen = jnp.where(keep, jnp.float32(0.0), jnp.float32(NEG_INF))
                news = list(acc[0:8])
                newm = list(acc[8:16])
                for node in range(16):
                    tj = _extract_f32(th, node)
                    wj = _extract_f32(wsum, node)
                    pj = _extract_f32(pen, node)
                    for q in range(8):
                        row = rows[j * 16 + node, pl.ds(q * 16, 16)]
                        news[q] = news[q] + row * wj
                        newm[q] = jnp.maximum(newm[q], row * tj + pj)
                acc = tuple(news) + tuple(newm) + (tie,)
            return acc

        accf = lax.fori_loop(0, mcount, _pool, acc0)
        denom = jnp.maximum(kk.astype(jnp.float32), jnp.float32(1.0))
        for q in range(8):
            outrow[pl.ds(q * 16, 16)] = accf[q] / denom
            mx = accf[8 + q]
            outrow[pl.ds(128 + q * 16, 16)] = jnp.where(
                mx > NEG_INF, mx, jnp.float32(0.0))
        pltpu.sync_copy(outrow, out_hbm.at[g])


@jax.jit
def kernel(x, edge_index, edge_attr, batch, W, b):
    h = _matvec(x, W)[:, 0]
    hp = jnp.pad(h, (0, NP - N))
    src = jnp.pad(edge_index[0], (0, EP - E)).reshape(NS, ECHUNKS, ECH)
    dst = jnp.pad(edge_index[1], (0, EP - E)).reshape(NS, ECHUNKS, ECH)
    w = jnp.pad(jnp.ones((E,), jnp.float32),
                (0, EP - E)).reshape(NS, ECHUNKS, ECH)
    bat = jnp.pad(batch, (0, NP - N), constant_values=G).reshape(NS, 5, 128)
    b16 = jnp.broadcast_to(b, (16,)).astype(jnp.float32)

    mesh = plsc.VectorSubcoreMesh(core_axis_name="c", subcore_axis_name="s",
                                  num_cores=1)
    out = pl.kernel(
        _sc_body,
        out_type=jax.ShapeDtypeStruct((G, 2 * D), jnp.float32),
        mesh=mesh,
        compiler_params=pltpu.CompilerParams(needs_layout_passes=False),
        scratch_types=[
            pltpu.VMEM_SHARED((NP,), jnp.float32),
            pltpu.VMEM_SHARED((NP,), jnp.float32),
            pltpu.VMEM_SHARED((NP,), jnp.float32),
            pltpu.VMEM_SHARED((80,), jnp.float32),
            pltpu.VMEM((ECHUNKS, ECH), jnp.int32),
            pltpu.VMEM((ECHUNKS, ECH), jnp.int32),
            pltpu.VMEM((ECHUNKS, ECH), jnp.float32),
            pltpu.VMEM((5, 128), jnp.int32),
            pltpu.VMEM((NPT,), jnp.float32),
            pltpu.VMEM((NP,), jnp.float32),
            pltpu.VMEM((NP,), jnp.float32),
            pltpu.VMEM((NP,), jnp.float32),
            pltpu.VMEM((80,), jnp.float32),
            pltpu.VMEM((80,), jnp.int32),
            pltpu.VMEM((128,), jnp.float32),
            pltpu.VMEM((64, D), jnp.float32),
            pltpu.VMEM((2 * D,), jnp.float32),
            pltpu.VMEM((16,), jnp.float32),
            pltpu.SemaphoreType.DMA,
        ],
    )(hp, src, dst, w, bat, b16, x)
    return out
